# Initial kernel scaffold; baseline (speedup 1.0000x reference)
import jax
import jax.numpy as jnp
from jax.experimental import pallas as pl


def kernel(batch):
    raise NotImplementedError("write your pallas kernel here")


# speedup vs baseline: 20.0152x; 20.0152x over previous
import numpy as np
import jax
import jax.numpy as jnp
from jax.experimental import pallas as pl
from jax.experimental.pallas import tpu as pltpu

_MIN_LOC = np.array([0.0, 0.0], dtype=np.float32)
_MAX_LOC = np.array([1.0, 1.0], dtype=np.float32)
_BINS = np.array([1023.0, 1023.0], dtype=np.float32)
_STRIDE = np.array([4.0, 4.0], dtype=np.float32)
_WINDOW = np.array([1.0, 1.0], dtype=np.float32)
_BATCH, _NLOC = 64, 200


def _build_base():
    window_side = (_WINDOW / 2.0).astype(np.int32).astype(np.float32)
    loc_delta = (_MAX_LOC - _MIN_LOC) / _BINS
    bins_window = _BINS - 2.0 * window_side
    min_window = _MIN_LOC + loc_delta * window_side
    max_window = _MIN_LOC + loc_delta * bins_window
    bins_stride = ((bins_window + 1.0) / _STRIDE).astype(np.int32)
    delta2 = (max_window - min_window) / bins_stride.astype(np.float32)
    xs = np.arange(min_window[0], max_window[0], delta2[0], dtype=np.float32)
    ys = np.arange(min_window[1], max_window[1], delta2[1], dtype=np.float32)
    X, Y = np.meshgrid(xs, ys)
    loc_base = np.stack([X, Y], axis=-1).astype(np.float32)
    corr_base = np.full(loc_base.shape, 0.634, dtype=np.float32)
    base4 = np.concatenate([corr_base, loc_base], axis=-1)
    return base4, loc_delta


_BASE4, _LOC_DELTA = _build_base()
_G = _BASE4.shape[0]
_DL0 = np.float32(_LOC_DELTA[0])
_DL1 = np.float32(_LOC_DELTA[1])
_BASE_TILED = _BASE4.reshape(_G, 8, 128)


def _map_body(batch_ref, base_ref, out_ref):
    out_ref[...] = base_ref[...][None]
    sub = jax.lax.broadcasted_iota(jnp.int32, (8, 128), 0)
    lane = jax.lax.broadcasted_iota(jnp.int32, (8, 128), 1)
    flat = sub * 128 + lane
    cell = flat // 4
    chan = flat % 4

    def loop(i, carry):
        l0 = batch_ref[0, i, 0]
        l1 = batch_ref[0, i, 1]
        px = (l0 / _DL0 / 4.0).astype(jnp.int32)
        py = (l1 / _DL1 / 4.0).astype(jnp.int32)
        rm = jnp.where(px < 1, px + (_G - 1), px - 1)
        cm = jnp.where(py < 1, py + (_G - 1), py - 1)
        vals = jnp.where(chan == 2, l0, jnp.where(chan == 3, l1, jnp.float32(1.0)))
        mask = (cell == cm) | (cell == py)
        t0 = out_ref[0, rm]
        out_ref[0, rm] = jnp.where(mask, vals, t0)
        t1 = out_ref[0, px]
        out_ref[0, px] = jnp.where(mask, vals, t1)
        return carry

    jax.lax.fori_loop(0, _NLOC, loop, 0)


def kernel(batch):
    base = jnp.asarray(_BASE_TILED)
    out = pl.pallas_call(
        _map_body,
        grid=(_BATCH,),
        in_specs=[
            pl.BlockSpec((1, _NLOC, 2), lambda b: (b, 0, 0)),
            pl.BlockSpec((_G, 8, 128), lambda b: (0, 0, 0)),
        ],
        out_specs=pl.BlockSpec((1, _G, 8, 128), lambda b: (b, 0, 0, 0)),
        out_shape=jax.ShapeDtypeStruct((_BATCH, _G, 8, 128), jnp.float32),
    )(batch, base)
    return out.reshape(_BATCH, _G, _G, 4)
